# baseline (speedup 1.0000x reference)
import functools

import jax
import jax.numpy as jnp
from jax import lax
from jax.experimental import pallas as pl
from jax.experimental.pallas import tpu as pltpu
from jax.experimental.pallas import tpu_sc as plsc

BATCH = 4
SEQ = 8192
DIM = 768
LANES = 16

NUM_CORES = 2
NUM_SUBCORES = 16
NW = NUM_CORES * NUM_SUBCORES
ROWS_PER_W = SEQ // NW
T = 32
NSTEP = ROWS_PER_W // T
NITER = NSTEP * BATCH
CVECS = DIM // LANES
RING = 3
PREF = 1


def _chunk_add(io_ref, tbl_ref):
    @plsc.parallel_loop(0, T)
    def _row(r):
        @plsc.parallel_loop(0, CVECS, unroll=8)
        def _col(c):
            sl = pl.ds(c * LANES, LANES)
            plsc.addupdate(io_ref.at[r, sl], tbl_ref[r, sl])


def _sc_body(in_hbm, tbl_hbm, out_hbm, tbl_v, io_v, *sems):
    s_tbl = sems[:2]
    s_in = sems[2:2 + RING]
    s_out = sems[2 + RING:2 + 2 * RING]

    wid = lax.axis_index("s") * NUM_CORES + lax.axis_index("c")
    base = wid * ROWS_PER_W

    def row0(step):
        return base + step * T

    def load_tbl(step):
        return pltpu.async_copy(
            tbl_hbm.at[pl.ds(row0(step), T)], tbl_v.at[step % 2],
            s_tbl[step % 2])

    def load_in(it):
        step, b = divmod(it, BATCH)
        return pltpu.async_copy(
            in_hbm.at[b, pl.ds(row0(step), T)], io_v.at[it % RING],
            s_in[it % RING])

    def store_out(it):
        step, b = divmod(it, BATCH)
        return pltpu.async_copy(
            io_v.at[it % RING], out_hbm.at[b, pl.ds(row0(step), T)],
            s_out[it % RING])

    h_tbl = [load_tbl(0), None]
    h_in = [None] * RING
    h_out = [None] * RING
    for it in range(PREF):
        h_in[it % RING] = load_in(it)

    for it in range(NITER):
        step, b = divmod(it, BATCH)
        nxt = it + PREF
        if nxt < NITER:
            if h_out[nxt % RING] is not None:
                h_out[nxt % RING].wait()
            h_in[nxt % RING] = load_in(nxt)
        if b == 0:
            if step + 1 < NSTEP:
                h_tbl[(step + 1) % 2] = load_tbl(step + 1)
            h_tbl[step % 2].wait()
        h_in[it % RING].wait()
        _chunk_add(io_v.at[it % RING], tbl_v.at[step % 2])
        h_out[it % RING] = store_out(it)

    for it in range(NITER - RING, NITER):
        h_out[it % RING].wait()


_sc_kernel = functools.partial(
    pl.kernel,
    out_type=jax.ShapeDtypeStruct((BATCH, SEQ, DIM), jnp.float32),
    mesh=plsc.VectorSubcoreMesh(core_axis_name="c", subcore_axis_name="s"),
    scratch_types=[
        pltpu.VMEM((2, T, DIM), jnp.float32),
        pltpu.VMEM((RING, T, DIM), jnp.float32),
    ] + [pltpu.SemaphoreType.DMA] * (2 + 2 * RING),
)(_sc_body)


def kernel(inputs, pos_table):
    return _sc_kernel(inputs, pos_table)


# speedup vs baseline: 1.0123x; 1.0123x over previous
import functools

import jax
import jax.numpy as jnp
from jax import lax
from jax.experimental import pallas as pl
from jax.experimental.pallas import tpu as pltpu
from jax.experimental.pallas import tpu_sc as plsc

BATCH = 4
SEQ = 8192
DIM = 768
LANES = 16

NUM_CORES = 2
NUM_SUBCORES = 16
NW = NUM_CORES * NUM_SUBCORES
ROWS_PER_W = SEQ // NW
T = 16
NSTEP = ROWS_PER_W // T
NITER = NSTEP * BATCH
CVECS = DIM // LANES
RING = 8
PREF = 4


def _chunk_add(io_ref, tbl_ref):
    @plsc.parallel_loop(0, T)
    def _row(r):
        @plsc.parallel_loop(0, CVECS, unroll=16)
        def _col(c):
            sl = pl.ds(c * LANES, LANES)
            plsc.addupdate(io_ref.at[r, sl], tbl_ref[r, sl])


def _sc_body(in_hbm, tbl_hbm, out_hbm, tbl_v, io_v, *sems):
    s_tbl = sems[:2]
    s_in = sems[2:2 + RING]
    s_out = sems[2 + RING:2 + 2 * RING]

    wid = lax.axis_index("s") * NUM_CORES + lax.axis_index("c")
    base = wid * ROWS_PER_W

    def row0(step):
        return base + step * T

    def load_tbl(step):
        return pltpu.async_copy(
            tbl_hbm.at[pl.ds(row0(step), T)], tbl_v.at[step % 2],
            s_tbl[step % 2])

    def load_in(it):
        step, b = divmod(it, BATCH)
        return pltpu.async_copy(
            in_hbm.at[b, pl.ds(row0(step), T)], io_v.at[it % RING],
            s_in[it % RING])

    def store_out(it):
        step, b = divmod(it, BATCH)
        return pltpu.async_copy(
            io_v.at[it % RING], out_hbm.at[b, pl.ds(row0(step), T)],
            s_out[it % RING])

    h_tbl = [load_tbl(0), None]
    h_in = [None] * RING
    h_out = [None] * RING
    for it in range(PREF):
        h_in[it % RING] = load_in(it)

    for it in range(NITER):
        step, b = divmod(it, BATCH)
        nxt = it + PREF
        if nxt < NITER:
            if h_out[nxt % RING] is not None:
                h_out[nxt % RING].wait()
            h_in[nxt % RING] = load_in(nxt)
        if b == 0:
            if step + 1 < NSTEP:
                h_tbl[(step + 1) % 2] = load_tbl(step + 1)
            h_tbl[step % 2].wait()
        h_in[it % RING].wait()
        _chunk_add(io_v.at[it % RING], tbl_v.at[step % 2])
        h_out[it % RING] = store_out(it)

    for it in range(NITER - RING, NITER):
        h_out[it % RING].wait()


_sc_kernel = functools.partial(
    pl.kernel,
    out_type=jax.ShapeDtypeStruct((BATCH, SEQ, DIM), jnp.float32),
    mesh=plsc.VectorSubcoreMesh(core_axis_name="c", subcore_axis_name="s"),
    scratch_types=[
        pltpu.VMEM((2, T, DIM), jnp.float32),
        pltpu.VMEM((RING, T, DIM), jnp.float32),
    ] + [pltpu.SemaphoreType.DMA] * (2 + 2 * RING),
)(_sc_body)


def kernel(inputs, pos_table):
    return _sc_kernel(inputs, pos_table)


# speedup vs baseline: 1.0397x; 1.0270x over previous
import functools

import jax
import jax.numpy as jnp
from jax import lax
from jax.experimental import pallas as pl
from jax.experimental.pallas import tpu as pltpu
from jax.experimental.pallas import tpu_sc as plsc

BATCH = 4
SEQ = 8192
DIM = 768
LANES = 16

NUM_CORES = 2
NUM_SUBCORES = 16
NW = NUM_CORES * NUM_SUBCORES
ROWS_PER_W = SEQ // NW
T = 16
NSTEP = ROWS_PER_W // T
NITER = NSTEP * BATCH
CVECS = DIM // LANES
RING = 8
PREF = 4


def _chunk_add(io_ref, tbl_ref):
    @plsc.parallel_loop(0, T)
    def _row(r):
        @plsc.parallel_loop(0, CVECS, unroll=8)
        def _col(c):
            sl = pl.ds(c * LANES, LANES)
            plsc.addupdate(io_ref.at[r, sl], tbl_ref[r, sl])


def _sc_body(in_hbm, tbl_hbm, out_hbm, tbl_v, io_v, *sems):
    s_tbl = sems[:2]
    s_in = sems[2:2 + RING]
    s_out = sems[2 + RING:2 + 2 * RING]

    wid = lax.axis_index("s") * NUM_CORES + lax.axis_index("c")
    base = wid * ROWS_PER_W

    def row0(step):
        return base + step * T

    def load_tbl(step):
        return pltpu.async_copy(
            tbl_hbm.at[pl.ds(row0(step), T)], tbl_v.at[step % 2],
            s_tbl[step % 2])

    def load_in(it):
        step, b = divmod(it, BATCH)
        return pltpu.async_copy(
            in_hbm.at[b, pl.ds(row0(step), T)], io_v.at[it % RING],
            s_in[it % RING])

    def store_out(it):
        step, b = divmod(it, BATCH)
        return pltpu.async_copy(
            io_v.at[it % RING], out_hbm.at[b, pl.ds(row0(step), T)],
            s_out[it % RING])

    h_tbl = [load_tbl(0), None]
    h_in = [None] * RING
    h_out = [None] * RING
    for it in range(PREF):
        h_in[it % RING] = load_in(it)

    for it in range(NITER):
        step, b = divmod(it, BATCH)
        nxt = it + PREF
        if nxt < NITER:
            if h_out[nxt % RING] is not None:
                h_out[nxt % RING].wait()
            h_in[nxt % RING] = load_in(nxt)
        if b == 0:
            if step + 1 < NSTEP:
                h_tbl[(step + 1) % 2] = load_tbl(step + 1)
            h_tbl[step % 2].wait()
        h_in[it % RING].wait()
        _chunk_add(io_v.at[it % RING], tbl_v.at[step % 2])
        h_out[it % RING] = store_out(it)

    for it in range(NITER - RING, NITER):
        h_out[it % RING].wait()


_sc_kernel = functools.partial(
    pl.kernel,
    out_type=jax.ShapeDtypeStruct((BATCH, SEQ, DIM), jnp.float32),
    mesh=plsc.VectorSubcoreMesh(core_axis_name="c", subcore_axis_name="s"),
    scratch_types=[
        pltpu.VMEM((2, T, DIM), jnp.float32),
        pltpu.VMEM((RING, T, DIM), jnp.float32),
    ] + [pltpu.SemaphoreType.DMA] * (2 + 2 * RING),
)(_sc_body)


def kernel(inputs, pos_table):
    return _sc_kernel(inputs, pos_table)
